# Initial kernel scaffold; baseline (speedup 1.0000x reference)
import functools

import jax
import jax.numpy as jnp
from jax import lax
from jax.experimental import pallas as pl
from jax.experimental.pallas import tpu as pltpu
from jax.experimental.pallas import tpu_sc as plsc

N = 10000
E = 320000
F_IN = 128
H = 32
W3 = 16
NC = 2
NS = 16
NW = NC * NS
IW = 128
ROWS = E // IW
RPW = ROWS // NW
TAIL = ROWS - RPW * NW
EPW = RPW * IW
K = 6
NCH = RPW // K
CE = K * IW
NPS = N // NS


def _seg_kernel(w_table, nscale):
    mesh = plsc.VectorSubcoreMesh(core_axis_name="c", subcore_axis_name="s")

    @functools.partial(
        pl.kernel,
        out_type=jax.ShapeDtypeStruct((NC, N, w_table), jnp.float32),
        mesh=mesh,
        scratch_types=[
            pltpu.VMEM((RPW, IW), jnp.int32),
            pltpu.VMEM((RPW, IW), jnp.int32),
            pltpu.VMEM((EPW,), jnp.float32),
            pltpu.VMEM((CE, w_table), jnp.float32),
            pltpu.VMEM((1, IW), jnp.int32),
            pltpu.VMEM((1, IW), jnp.int32),
            pltpu.VMEM((IW,), jnp.float32),
            pltpu.VMEM((IW, w_table), jnp.float32),
            pltpu.VMEM_SHARED((N, w_table), jnp.float32),
            pltpu.SemaphoreType.DMA,
        ],
    )
    def seg(y_hbm, src_hbm, dst_hbm, ew_hbm, zero_hbm, out_hbm,
            src_v, dst_v, ew_v, rows_v, tsrc_v, tdst_v, tew_v, trows_v,
            acc, sem):
        c = lax.axis_index("c")
        s = lax.axis_index("s")
        w = c * NS + s

        pltpu.sync_copy(zero_hbm.at[pl.ds(s * NPS, NPS)],
                        acc.at[pl.ds(s * NPS, NPS)])

        pltpu.sync_copy(src_hbm.at[pl.ds(w * RPW, RPW)], src_v)
        pltpu.sync_copy(dst_hbm.at[pl.ds(w * RPW, RPW)], dst_v)
        pltpu.sync_copy(ew_hbm.at[pl.ds(w * EPW, EPW)], ew_v)

        plsc.subcore_barrier()

        def scale_rows(buf, ewbuf, eoff, g):
            e16 = ewbuf[pl.ds(eoff + g * 16, 16)]
            ridx = g * 16 + lax.iota(jnp.int32, 16)
            for j in range(nscale):
                cidx = jnp.full((16,), j, jnp.int32)
                vals = plsc.load_gather(buf, [ridx, cidx])
                plsc.store_scatter(buf, [ridx, cidx], vals * e16)

        def chunk_body(ci, carry):
            r0 = ci * K
            cps = [
                pltpu.async_copy(
                    y_hbm.at[src_v.at[r0 + kk]],
                    rows_v.at[pl.ds(kk * IW, IW)], sem)
                for kk in range(K)
            ]
            for cp in cps:
                cp.wait()

            def scale_body(g, carry2):
                scale_rows(rows_v, ew_v, ci * CE, g)
                return carry2
            lax.fori_loop(0, CE // 16, scale_body, 0)

            for kk in range(K):
                pltpu.sync_copy(rows_v.at[pl.ds(kk * IW, IW)],
                                acc.at[dst_v.at[r0 + kk]], add=True)
            return carry
        lax.fori_loop(0, NCH, chunk_body, 0)

        @pl.when(w < TAIL)
        def _():
            tr = ROWS - TAIL + w
            pltpu.sync_copy(src_hbm.at[pl.ds(tr, 1)], tsrc_v)
            pltpu.sync_copy(dst_hbm.at[pl.ds(tr, 1)], tdst_v)
            pltpu.sync_copy(ew_hbm.at[pl.ds(tr * IW, IW)], tew_v)
            pltpu.async_copy(y_hbm.at[tsrc_v.at[0]], trows_v, sem).wait()

            def tscale_body(g, carry2):
                scale_rows(trows_v, tew_v, 0, g)
                return carry2
            lax.fori_loop(0, IW // 16, tscale_body, 0)
            pltpu.sync_copy(trows_v, acc.at[tdst_v.at[0]], add=True)

        plsc.subcore_barrier()
        pltpu.sync_copy(acc.at[pl.ds(s * NPS, NPS)],
                        out_hbm.at[c, pl.ds(s * NPS, NPS)])

    return seg


_seg32 = _seg_kernel(H, H)
_seg16 = _seg_kernel(W3, 2)

_BLK = 2000


def _dense_in(x, w_rel, w_root, b_rel):
    def body(x_ref, wr_ref, wo_ref, b_ref, y_ref, r_ref):
        xb = x_ref[...]
        y_ref[...] = jnp.dot(xb, wr_ref[...], preferred_element_type=jnp.float32)
        r_ref[...] = (jnp.dot(xb, wo_ref[...], preferred_element_type=jnp.float32)
                      + b_ref[...])
    return pl.pallas_call(
        body,
        grid=(N // _BLK,),
        in_specs=[
            pl.BlockSpec((_BLK, F_IN), lambda i: (i, 0)),
            pl.BlockSpec((F_IN, H), lambda i: (0, 0)),
            pl.BlockSpec((F_IN, H), lambda i: (0, 0)),
            pl.BlockSpec((1, H), lambda i: (0, 0)),
        ],
        out_specs=[
            pl.BlockSpec((_BLK, H), lambda i: (i, 0)),
            pl.BlockSpec((_BLK, H), lambda i: (i, 0)),
        ],
        out_shape=[jax.ShapeDtypeStruct((N, H), jnp.float32)] * 2,
    )(x, w_rel, w_root, b_rel.reshape(1, H))


def _dense_hidden(parts, r_in, w_rel, w_root, b_rel):
    def body(p_ref, ri_ref, wr_ref, wo_ref, b_ref, y_ref, r_ref):
        h = jnp.maximum(p_ref[0] + p_ref[1] + ri_ref[...], 0.0)
        y_ref[...] = jnp.dot(h, wr_ref[...], preferred_element_type=jnp.float32)
        r_ref[...] = (jnp.dot(h, wo_ref[...], preferred_element_type=jnp.float32)
                      + b_ref[...])
    return pl.pallas_call(
        body,
        grid=(N // _BLK,),
        in_specs=[
            pl.BlockSpec((NC, _BLK, H), lambda i: (0, i, 0)),
            pl.BlockSpec((_BLK, H), lambda i: (i, 0)),
            pl.BlockSpec((H, H), lambda i: (0, 0)),
            pl.BlockSpec((H, H), lambda i: (0, 0)),
            pl.BlockSpec((1, H), lambda i: (0, 0)),
        ],
        out_specs=[
            pl.BlockSpec((_BLK, H), lambda i: (i, 0)),
            pl.BlockSpec((_BLK, H), lambda i: (i, 0)),
        ],
        out_shape=[jax.ShapeDtypeStruct((N, H), jnp.float32)] * 2,
    )(parts, r_in, w_rel, w_root, b_rel.reshape(1, H))


def _dense_pv(parts, r_in, w_rel_pv, w_root_pv, b_pv, cn):
    def body(cn_ref, p_ref, ri_ref, wr_ref, wo_ref, b_ref, y_ref, r_ref):
        i = pl.program_id(0)
        h = jnp.maximum(p_ref[0] + p_ref[1] + ri_ref[...], 0.0)
        row = lax.broadcasted_iota(jnp.int32, (_BLK, W3), 0) + i * _BLK
        col = lax.broadcasted_iota(jnp.int32, (_BLK, W3), 1)
        ind = jnp.where((row == cn_ref[0]) & (col == 2), 1.0, 0.0)
        y_ref[...] = (jnp.dot(h, wr_ref[...], preferred_element_type=jnp.float32)
                      + ind)
        r_ref[...] = (jnp.dot(h, wo_ref[...], preferred_element_type=jnp.float32)
                      + b_ref[...])
    return pl.pallas_call(
        body,
        grid=(N // _BLK,),
        in_specs=[
            pl.BlockSpec(memory_space=pltpu.SMEM),
            pl.BlockSpec((NC, _BLK, H), lambda i: (0, i, 0)),
            pl.BlockSpec((_BLK, H), lambda i: (i, 0)),
            pl.BlockSpec((H, W3), lambda i: (0, 0)),
            pl.BlockSpec((H, W3), lambda i: (0, 0)),
            pl.BlockSpec((1, W3), lambda i: (0, 0)),
        ],
        out_specs=[
            pl.BlockSpec((_BLK, W3), lambda i: (i, 0)),
            pl.BlockSpec((_BLK, W3), lambda i: (i, 0)),
        ],
        out_shape=[jax.ShapeDtypeStruct((N, W3), jnp.float32)] * 2,
    )(cn, parts, r_in, w_rel_pv, w_root_pv, b_pv.reshape(1, W3))


def _final(parts, rpv):
    def body(p_ref, rpv_ref, pt_ref, vt_ref):
        s = p_ref[0] + p_ref[1] + rpv_ref[...]
        p = s[:, 0:1]
        v = s[:, 1:2]
        mask = jnp.where(s[:, 2:3] > 0.0, 1.0, 0.0)
        pt = mask * p
        pt = jnp.where(pt == 0.0, -jnp.inf, pt)
        m = jnp.max(pt)
        ex = jnp.exp(pt - m)
        pt_ref[...] = ex / jnp.sum(ex)
        vt_ref[...] = mask * v
    return pl.pallas_call(
        body,
        in_specs=[
            pl.BlockSpec((NC, N, W3), lambda: (0, 0, 0)),
            pl.BlockSpec((N, W3), lambda: (0, 0)),
        ],
        out_specs=[
            pl.BlockSpec((N, 1), lambda: (0, 0)),
            pl.BlockSpec((N, 1), lambda: (0, 0)),
        ],
        out_shape=[jax.ShapeDtypeStruct((N, 1), jnp.float32)] * 2,
    )(parts, rpv)


def kernel(x, edge_index, edge_attr, current_node,
           W_rel_in, b_rel_in, W_root_in,
           W_rel_h, b_rel_h, W_root_h,
           W_rel_p, b_rel_p, W_root_p,
           W_rel_v, b_rel_v, W_root_v):
    src2d = edge_index[0].reshape(ROWS, IW)
    dst2d = edge_index[1].reshape(ROWS, IW)
    ew = edge_attr
    zeros_h = jnp.zeros((N, H), jnp.float32)
    zeros_w3 = jnp.zeros((N, W3), jnp.float32)
    cn = jnp.asarray(current_node, jnp.int32).reshape(1)

    w_rel_pv = jnp.zeros((H, W3), jnp.float32)
    w_rel_pv = w_rel_pv.at[:, 0:1].set(W_rel_p).at[:, 1:2].set(W_rel_v)
    w_root_pv = jnp.zeros((H, W3), jnp.float32)
    w_root_pv = w_root_pv.at[:, 0:1].set(W_root_p).at[:, 1:2].set(W_root_v)
    b_pv = jnp.zeros((W3,), jnp.float32)
    b_pv = b_pv.at[0].set(b_rel_p[0]).at[1].set(b_rel_v[0])

    y1, r1 = _dense_in(x, W_rel_in, W_root_in, b_rel_in)
    parts1 = _seg32(y1, src2d, dst2d, ew, zeros_h)
    y2, r2 = _dense_hidden(parts1, r1, W_rel_h, W_root_h, b_rel_h)
    parts2 = _seg32(y2, src2d, dst2d, ew, zeros_h)
    ypv, rpv = _dense_pv(parts2, r2, w_rel_pv, w_root_pv, b_pv, cn)
    parts3 = _seg16(ypv, src2d, dst2d, ew, zeros_w3)
    pt, vt = _final(parts3, rpv)
    return pt.reshape(-1), vt.reshape(-1)


# speedup vs baseline: 1.9252x; 1.9252x over previous
import functools

import jax
import jax.numpy as jnp
from jax import lax
from jax.experimental import pallas as pl
from jax.experimental.pallas import tpu as pltpu
from jax.experimental.pallas import tpu_sc as plsc

N = 10000
E = 320000
F_IN = 128
H = 32
W3 = 40
NC = 2
NS = 16
NW = NC * NS
IW = 128
ROWS = E // IW
RPW = ROWS // NW
TAIL = ROWS - RPW * NW
EPW = RPW * IW
NPS = N // NS


def _seg_kernel(w_table, nscale, k_chunk, upfront):
    mesh = plsc.VectorSubcoreMesh(core_axis_name="c", subcore_axis_name="s",
                                  num_cores=NC, num_subcores=NS)
    nch = RPW // k_chunk
    ce = k_chunk * IW
    if upfront:
        idx_scr = [
            pltpu.VMEM((RPW + 1, IW), jnp.int32),
            pltpu.VMEM((RPW + 1, IW), jnp.int32),
            pltpu.VMEM((EPW + IW,), jnp.float32),
        ]
    else:
        idx_scr = [
            pltpu.VMEM((k_chunk, IW), jnp.int32),
            pltpu.VMEM((k_chunk, IW), jnp.int32),
            pltpu.VMEM((k_chunk * IW,), jnp.float32),
        ]

    @functools.partial(
        pl.kernel,
        out_type=jax.ShapeDtypeStruct((NC, N, w_table), jnp.float32),
        mesh=mesh,
        compiler_params=pltpu.CompilerParams(use_tc_tiling_on_sc=False,
                                             needs_layout_passes=False),
        scratch_types=idx_scr + [
            pltpu.VMEM((ce, w_table), jnp.float32),
            pltpu.VMEM_SHARED((N, w_table), jnp.float32),
            pltpu.SemaphoreType.DMA,
        ],
    )
    def seg(y_hbm, src_hbm, dst_hbm, ew_hbm, zero_hbm, out_hbm,
            src_v, dst_v, ew_v, rows_v, acc, sem):
        c = lax.axis_index("c")
        s = lax.axis_index("s")
        w = c * NS + s

        pltpu.sync_copy(zero_hbm.at[pl.ds(s * NPS, NPS)],
                        acc.at[pl.ds(s * NPS, NPS)])

        if upfront:
            pltpu.sync_copy(src_hbm.at[pl.ds(w * RPW, RPW)],
                            src_v.at[pl.ds(0, RPW)])
            pltpu.sync_copy(dst_hbm.at[pl.ds(w * RPW, RPW)],
                            dst_v.at[pl.ds(0, RPW)])
            pltpu.sync_copy(ew_hbm.at[pl.ds(w * EPW, EPW)],
                            ew_v.at[pl.ds(0, EPW)])

            @pl.when(w < TAIL)
            def _():
                tr = ROWS - TAIL + w
                pltpu.sync_copy(src_hbm.at[pl.ds(tr, 1)],
                                src_v.at[pl.ds(RPW, 1)])
                pltpu.sync_copy(dst_hbm.at[pl.ds(tr, 1)],
                                dst_v.at[pl.ds(RPW, 1)])
                pltpu.sync_copy(ew_hbm.at[pl.ds(tr * IW, IW)],
                                ew_v.at[pl.ds(EPW, IW)])

        plsc.subcore_barrier()

        def scale_row(row_base, eoff_row):
            def grp(g, _):
                e16 = ew_v[pl.ds(eoff_row + g * 16, 16)]
                ridx = row_base + g * 16 + lax.iota(jnp.int32, 16)
                for j in range(nscale):
                    cidx = jnp.full((16,), j, jnp.int32)
                    vals = plsc.load_gather(rows_v, [ridx, cidx])
                    plsc.store_scatter(rows_v, [ridx, cidx], vals * e16)
                return 0
            lax.fori_loop(0, 8, grp, 0)

        def run_rows(src_ref, dst_ref, row0, n_static, eoff0):
            cps = [
                pltpu.async_copy(
                    y_hbm.at[src_ref.at[row0 + kk]],
                    rows_v.at[pl.ds(kk * IW, IW)], sem)
                for kk in range(n_static)
            ]
            for cp in cps:
                cp.wait()
            for kk in range(n_static):
                scale_row(kk * IW, eoff0 + kk * IW)
            for kk in range(n_static):
                pltpu.sync_copy(rows_v.at[pl.ds(kk * IW, IW)],
                                acc.at[dst_ref.at[row0 + kk]], add=True)

        def chunk_body(ci, carry):
            if upfront:
                run_rows(src_v, dst_v, ci * k_chunk, k_chunk, ci * ce)
            else:
                r0 = w * RPW + ci * k_chunk
                pltpu.sync_copy(src_hbm.at[pl.ds(r0, k_chunk)], src_v)
                pltpu.sync_copy(dst_hbm.at[pl.ds(r0, k_chunk)], dst_v)
                pltpu.sync_copy(ew_hbm.at[pl.ds(r0 * IW, ce)], ew_v)
                run_rows(src_v, dst_v, 0, k_chunk, 0)
            return carry
        lax.fori_loop(0, nch, chunk_body, 0)

        @pl.when(w < TAIL)
        def _():
            if upfront:
                run_rows(src_v, dst_v, RPW, 1, EPW)
            else:
                tr = ROWS - TAIL + w
                pltpu.sync_copy(src_hbm.at[pl.ds(tr, 1)],
                                src_v.at[pl.ds(0, 1)])
                pltpu.sync_copy(dst_hbm.at[pl.ds(tr, 1)],
                                dst_v.at[pl.ds(0, 1)])
                pltpu.sync_copy(ew_hbm.at[pl.ds(tr * IW, IW)],
                                ew_v.at[pl.ds(0, IW)])
                run_rows(src_v, dst_v, 0, 1, 0)

        plsc.subcore_barrier()
        pltpu.sync_copy(acc.at[pl.ds(s * NPS, NPS)],
                        out_hbm.at[c, pl.ds(s * NPS, NPS)])

    return seg


@functools.lru_cache(maxsize=None)
def _seg1():
    return _seg_kernel(F_IN, F_IN, 2, False)


@functools.lru_cache(maxsize=None)
def _seg2():
    return _seg_kernel(H, H, 6, True)


@functools.lru_cache(maxsize=None)
def _seg3():
    return _seg_kernel(W3, H, 6, True)


_BLK = 2000


def _layer1(parts, x, w_rel, w_root, b_rel):
    def body(p_ref, x_ref, wr_ref, wo_ref, b_ref, h_ref):
        agg = p_ref[0] + p_ref[1]
        h_ref[...] = jnp.maximum(
            jnp.dot(agg, wr_ref[...], preferred_element_type=jnp.float32)
            + b_ref[...]
            + jnp.dot(x_ref[...], wo_ref[...],
                      preferred_element_type=jnp.float32), 0.0)
    return pl.pallas_call(
        body,
        grid=(N // _BLK,),
        in_specs=[
            pl.BlockSpec((NC, _BLK, F_IN), lambda i: (0, i, 0)),
            pl.BlockSpec((_BLK, F_IN), lambda i: (i, 0)),
            pl.BlockSpec((F_IN, H), lambda i: (0, 0)),
            pl.BlockSpec((F_IN, H), lambda i: (0, 0)),
            pl.BlockSpec((1, H), lambda i: (0, 0)),
        ],
        out_specs=pl.BlockSpec((_BLK, H), lambda i: (i, 0)),
        out_shape=jax.ShapeDtypeStruct((N, H), jnp.float32),
    )(parts, x, w_rel, w_root, b_rel.reshape(1, H))


def _layer2(parts, h1, w_rel, w_root, b_rel, cn):
    def body(cn_ref, p_ref, h_ref, wr_ref, wo_ref, b_ref, t_ref):
        i = pl.program_id(0)
        agg = p_ref[0] + p_ref[1]
        h2 = jnp.maximum(
            jnp.dot(agg, wr_ref[...], preferred_element_type=jnp.float32)
            + b_ref[...]
            + jnp.dot(h_ref[...], wo_ref[...],
                      preferred_element_type=jnp.float32), 0.0)
        row = lax.broadcasted_iota(jnp.int32, (_BLK, W3 - H), 0) + i * _BLK
        col = lax.broadcasted_iota(jnp.int32, (_BLK, W3 - H), 1)
        ind = jnp.where((row == cn_ref[0]) & (col == 0), 1.0, 0.0)
        t_ref[...] = jnp.concatenate([h2, ind], axis=1)
    return pl.pallas_call(
        body,
        grid=(N // _BLK,),
        in_specs=[
            pl.BlockSpec(memory_space=pltpu.SMEM),
            pl.BlockSpec((NC, _BLK, H), lambda i: (0, i, 0)),
            pl.BlockSpec((_BLK, H), lambda i: (i, 0)),
            pl.BlockSpec((H, H), lambda i: (0, 0)),
            pl.BlockSpec((H, H), lambda i: (0, 0)),
            pl.BlockSpec((1, H), lambda i: (0, 0)),
        ],
        out_specs=pl.BlockSpec((_BLK, W3), lambda i: (i, 0)),
        out_shape=jax.ShapeDtypeStruct((N, W3), jnp.float32),
    )(cn, parts, h1, w_rel, w_root, b_rel.reshape(1, H))


def _final(parts, tab, w_rel_p, w_root_p, b_p, w_rel_v, w_root_v, b_v):
    def body(p_ref, t_ref, wrp_ref, wop_ref, bp_ref, wrv_ref, wov_ref,
             bv_ref, pt_ref, vt_ref):
        ssum = p_ref[0] + p_ref[1]
        agg = ssum[:, :H]
        cnt = ssum[:, H:H + 1]
        h2 = t_ref[:, :H]
        p = (jnp.dot(agg, wrp_ref[...], preferred_element_type=jnp.float32)
             + bp_ref[0]
             + jnp.dot(h2, wop_ref[...], preferred_element_type=jnp.float32))
        v = (jnp.dot(agg, wrv_ref[...], preferred_element_type=jnp.float32)
             + bv_ref[0]
             + jnp.dot(h2, wov_ref[...], preferred_element_type=jnp.float32))
        mask = jnp.where(cnt > 0.0, 1.0, 0.0)
        pt = mask * p
        pt = jnp.where(pt == 0.0, -jnp.inf, pt)
        m = jnp.max(pt)
        ex = jnp.exp(pt - m)
        pt_ref[...] = ex / jnp.sum(ex)
        vt_ref[...] = mask * v
    return pl.pallas_call(
        body,
        in_specs=[
            pl.BlockSpec((NC, N, W3), lambda: (0, 0, 0)),
            pl.BlockSpec((N, W3), lambda: (0, 0)),
            pl.BlockSpec((H, 1), lambda: (0, 0)),
            pl.BlockSpec((H, 1), lambda: (0, 0)),
            pl.BlockSpec(memory_space=pltpu.SMEM),
            pl.BlockSpec((H, 1), lambda: (0, 0)),
            pl.BlockSpec((H, 1), lambda: (0, 0)),
            pl.BlockSpec(memory_space=pltpu.SMEM),
        ],
        out_specs=[
            pl.BlockSpec((N, 1), lambda: (0, 0)),
            pl.BlockSpec((N, 1), lambda: (0, 0)),
        ],
        out_shape=[jax.ShapeDtypeStruct((N, 1), jnp.float32)] * 2,
    )(parts, tab, w_rel_p, w_root_p, b_p, w_rel_v, w_root_v, b_v)


def kernel(x, edge_index, edge_attr, current_node,
           W_rel_in, b_rel_in, W_root_in,
           W_rel_h, b_rel_h, W_root_h,
           W_rel_p, b_rel_p, W_root_p,
           W_rel_v, b_rel_v, W_root_v):
    src2d = edge_index[0].reshape(ROWS, IW)
    dst2d = edge_index[1].reshape(ROWS, IW)
    ew = edge_attr
    zeros_f = jnp.zeros((N, F_IN), jnp.float32)
    zeros_h = jnp.zeros((N, H), jnp.float32)
    zeros_w3 = jnp.zeros((N, W3), jnp.float32)
    cn = jnp.asarray(current_node, jnp.int32).reshape(1)

    parts1 = _seg1()(x, src2d, dst2d, ew, zeros_f)
    h1 = _layer1(parts1, x, W_rel_in, W_root_in, b_rel_in)
    parts2 = _seg2()(h1, src2d, dst2d, ew, zeros_h)
    tab3 = _layer2(parts2, h1, W_rel_h, W_root_h, b_rel_h, cn)
    parts3 = _seg3()(tab3, src2d, dst2d, ew, zeros_w3)
    pt, vt = _final(parts3, tab3, W_rel_p, W_root_p, b_rel_p,
                    W_rel_v, W_root_v, b_rel_v)
    return pt.reshape(-1), vt.reshape(-1)


# speedup vs baseline: 3.6325x; 1.8868x over previous
import functools

import jax
import jax.numpy as jnp
from jax import lax
from jax.experimental import pallas as pl
from jax.experimental.pallas import tpu as pltpu
from jax.experimental.pallas import tpu_sc as plsc

N = 10000
E = 320000
F_IN = 128
H = 32
W3 = 40
NC = 2
NS = 16
NW = NC * NS
IW = 128
ROWS = E // IW
RPW = ROWS // NW
TAIL = ROWS - RPW * NW
EPW = RPW * IW
NPS = N // NS


def _seg_kernel(w_table, nscale, k_chunk, upfront):
    mesh = plsc.VectorSubcoreMesh(core_axis_name="c", subcore_axis_name="s",
                                  num_cores=NC, num_subcores=NS)
    nch = RPW // k_chunk
    ce = k_chunk * IW
    if upfront:
        idx_scr = [
            pltpu.VMEM((RPW + 1, IW), jnp.int32),
            pltpu.VMEM((RPW + 1, IW), jnp.int32),
        ]
    else:
        idx_scr = [
            pltpu.VMEM((k_chunk, IW), jnp.int32),
            pltpu.VMEM((k_chunk, IW), jnp.int32),
        ]
    idx_scr.append(pltpu.VMEM((ce * 16,), jnp.float32))

    @functools.partial(
        pl.kernel,
        out_type=jax.ShapeDtypeStruct((NC, N, w_table), jnp.float32),
        mesh=mesh,
        compiler_params=pltpu.CompilerParams(use_tc_tiling_on_sc=False,
                                             needs_layout_passes=False),
        scratch_types=idx_scr + [
            pltpu.VMEM((ce, w_table), jnp.float32),
            pltpu.VMEM_SHARED((N, w_table), jnp.float32),
            pltpu.SemaphoreType.DMA,
        ],
    )
    def seg(y_hbm, src_hbm, dst_hbm, ew16_hbm, zero_hbm, out_hbm,
            src_v, dst_v, ew16_v, rows_v, acc, sem):
        c = lax.axis_index("c")
        s = lax.axis_index("s")
        w = c * NS + s

        pltpu.sync_copy(zero_hbm.at[pl.ds(s * NPS, NPS)],
                        acc.at[pl.ds(s * NPS, NPS)])

        if upfront:
            pltpu.sync_copy(src_hbm.at[pl.ds(w * RPW, RPW)],
                            src_v.at[pl.ds(0, RPW)])
            pltpu.sync_copy(dst_hbm.at[pl.ds(w * RPW, RPW)],
                            dst_v.at[pl.ds(0, RPW)])

            @pl.when(w < TAIL)
            def _():
                tr = ROWS - TAIL + w
                pltpu.sync_copy(src_hbm.at[pl.ds(tr, 1)],
                                src_v.at[pl.ds(RPW, 1)])
                pltpu.sync_copy(dst_hbm.at[pl.ds(tr, 1)],
                                dst_v.at[pl.ds(RPW, 1)])

        plsc.subcore_barrier()

        def scale_row(row_base):
            def edge_body(e, _):
                loc = row_base + e
                e16 = ew16_v[pl.ds(loc * 16, 16)]
                ridx = loc + jnp.zeros((16,), jnp.int32)
                for hw in range(nscale // 16):
                    cidx = hw * 16 + lax.iota(jnp.int32, 16)
                    vals = plsc.load_gather(rows_v, [ridx, cidx])
                    plsc.store_scatter(rows_v, [ridx, cidx], vals * e16)
                return 0
            lax.fori_loop(0, IW, edge_body, 0)

        def run_rows(src_ref, dst_ref, row0, n_static):
            cps = [
                pltpu.async_copy(
                    y_hbm.at[src_ref.at[row0 + kk]],
                    rows_v.at[pl.ds(kk * IW, IW)], sem)
                for kk in range(n_static)
            ]
            for cp in cps:
                cp.wait()
            for kk in range(n_static):
                scale_row(kk * IW)
            for kk in range(n_static):
                pltpu.sync_copy(rows_v.at[pl.ds(kk * IW, IW)],
                                acc.at[dst_ref.at[row0 + kk]], add=True)

        def chunk_body(ci, carry):
            e0 = w * EPW + ci * ce
            pltpu.sync_copy(ew16_hbm.at[pl.ds(e0 * 16, ce * 16)], ew16_v)
            if upfront:
                run_rows(src_v, dst_v, ci * k_chunk, k_chunk)
            else:
                r0 = w * RPW + ci * k_chunk
                pltpu.sync_copy(src_hbm.at[pl.ds(r0, k_chunk)], src_v)
                pltpu.sync_copy(dst_hbm.at[pl.ds(r0, k_chunk)], dst_v)
                run_rows(src_v, dst_v, 0, k_chunk)
            return carry
        lax.fori_loop(0, nch, chunk_body, 0)

        @pl.when(w < TAIL)
        def _():
            tr = ROWS - TAIL + w
            pltpu.sync_copy(ew16_hbm.at[pl.ds(tr * IW * 16, IW * 16)],
                            ew16_v.at[pl.ds(0, IW * 16)])
            if upfront:
                run_rows(src_v, dst_v, RPW, 1)
            else:
                pltpu.sync_copy(src_hbm.at[pl.ds(tr, 1)],
                                src_v.at[pl.ds(0, 1)])
                pltpu.sync_copy(dst_hbm.at[pl.ds(tr, 1)],
                                dst_v.at[pl.ds(0, 1)])
                run_rows(src_v, dst_v, 0, 1)

        plsc.subcore_barrier()
        pltpu.sync_copy(acc.at[pl.ds(s * NPS, NPS)],
                        out_hbm.at[c, pl.ds(s * NPS, NPS)])

    return seg


@functools.lru_cache(maxsize=None)
def _seg1():
    return _seg_kernel(F_IN, F_IN, 2, False)


@functools.lru_cache(maxsize=None)
def _seg2():
    return _seg_kernel(H, H, 6, True)


@functools.lru_cache(maxsize=None)
def _seg3():
    return _seg_kernel(W3, H, 6, True)


_BLK = 2000


def _layer1(parts, x, w_rel, w_root, b_rel):
    def body(p_ref, x_ref, wr_ref, wo_ref, b_ref, h_ref):
        agg = p_ref[0] + p_ref[1]
        h_ref[...] = jnp.maximum(
            jnp.dot(agg, wr_ref[...], preferred_element_type=jnp.float32)
            + b_ref[...]
            + jnp.dot(x_ref[...], wo_ref[...],
                      preferred_element_type=jnp.float32), 0.0)
    return pl.pallas_call(
        body,
        grid=(N // _BLK,),
        in_specs=[
            pl.BlockSpec((NC, _BLK, F_IN), lambda i: (0, i, 0)),
            pl.BlockSpec((_BLK, F_IN), lambda i: (i, 0)),
            pl.BlockSpec((F_IN, H), lambda i: (0, 0)),
            pl.BlockSpec((F_IN, H), lambda i: (0, 0)),
            pl.BlockSpec((1, H), lambda i: (0, 0)),
        ],
        out_specs=pl.BlockSpec((_BLK, H), lambda i: (i, 0)),
        out_shape=jax.ShapeDtypeStruct((N, H), jnp.float32),
    )(parts, x, w_rel, w_root, b_rel.reshape(1, H))


def _layer2(parts, h1, w_rel, w_root, b_rel, cn):
    def body(cn_ref, p_ref, h_ref, wr_ref, wo_ref, b_ref, t_ref):
        i = pl.program_id(0)
        agg = p_ref[0] + p_ref[1]
        h2 = jnp.maximum(
            jnp.dot(agg, wr_ref[...], preferred_element_type=jnp.float32)
            + b_ref[...]
            + jnp.dot(h_ref[...], wo_ref[...],
                      preferred_element_type=jnp.float32), 0.0)
        row = lax.broadcasted_iota(jnp.int32, (_BLK, W3 - H), 0) + i * _BLK
        col = lax.broadcasted_iota(jnp.int32, (_BLK, W3 - H), 1)
        ind = jnp.where((row == cn_ref[0]) & (col == 0), 1.0, 0.0)
        t_ref[...] = jnp.concatenate([h2, ind], axis=1)
    return pl.pallas_call(
        body,
        grid=(N // _BLK,),
        in_specs=[
            pl.BlockSpec(memory_space=pltpu.SMEM),
            pl.BlockSpec((NC, _BLK, H), lambda i: (0, i, 0)),
            pl.BlockSpec((_BLK, H), lambda i: (i, 0)),
            pl.BlockSpec((H, H), lambda i: (0, 0)),
            pl.BlockSpec((H, H), lambda i: (0, 0)),
            pl.BlockSpec((1, H), lambda i: (0, 0)),
        ],
        out_specs=pl.BlockSpec((_BLK, W3), lambda i: (i, 0)),
        out_shape=jax.ShapeDtypeStruct((N, W3), jnp.float32),
    )(cn, parts, h1, w_rel, w_root, b_rel.reshape(1, H))


def _final(parts, tab, w_rel_p, w_root_p, b_p, w_rel_v, w_root_v, b_v):
    def body(p_ref, t_ref, wrp_ref, wop_ref, bp_ref, wrv_ref, wov_ref,
             bv_ref, pt_ref, vt_ref):
        ssum = p_ref[0] + p_ref[1]
        agg = ssum[:, :H]
        cnt = ssum[:, H:H + 1]
        h2 = t_ref[:, :H]
        p = (jnp.dot(agg, wrp_ref[...], preferred_element_type=jnp.float32)
             + bp_ref[0]
             + jnp.dot(h2, wop_ref[...], preferred_element_type=jnp.float32))
        v = (jnp.dot(agg, wrv_ref[...], preferred_element_type=jnp.float32)
             + bv_ref[0]
             + jnp.dot(h2, wov_ref[...], preferred_element_type=jnp.float32))
        mask = jnp.where(cnt > 0.0, 1.0, 0.0)
        pt = mask * p
        pt = jnp.where(pt == 0.0, -jnp.inf, pt)
        m = jnp.max(pt)
        ex = jnp.exp(pt - m)
        pt_ref[...] = ex / jnp.sum(ex)
        vt_ref[...] = mask * v
    return pl.pallas_call(
        body,
        in_specs=[
            pl.BlockSpec((NC, N, W3), lambda: (0, 0, 0)),
            pl.BlockSpec((N, W3), lambda: (0, 0)),
            pl.BlockSpec((H, 1), lambda: (0, 0)),
            pl.BlockSpec((H, 1), lambda: (0, 0)),
            pl.BlockSpec(memory_space=pltpu.SMEM),
            pl.BlockSpec((H, 1), lambda: (0, 0)),
            pl.BlockSpec((H, 1), lambda: (0, 0)),
            pl.BlockSpec(memory_space=pltpu.SMEM),
        ],
        out_specs=[
            pl.BlockSpec((N, 1), lambda: (0, 0)),
            pl.BlockSpec((N, 1), lambda: (0, 0)),
        ],
        out_shape=[jax.ShapeDtypeStruct((N, 1), jnp.float32)] * 2,
    )(parts, tab, w_rel_p, w_root_p, b_p, w_rel_v, w_root_v, b_v)


def kernel(x, edge_index, edge_attr, current_node,
           W_rel_in, b_rel_in, W_root_in,
           W_rel_h, b_rel_h, W_root_h,
           W_rel_p, b_rel_p, W_root_p,
           W_rel_v, b_rel_v, W_root_v):
    src2d = edge_index[0].reshape(ROWS, IW)
    dst2d = edge_index[1].reshape(ROWS, IW)
    ew16 = jnp.broadcast_to(edge_attr.reshape(E, 1), (E, 16)).reshape(E * 16)
    zeros_f = jnp.zeros((N, F_IN), jnp.float32)
    zeros_h = jnp.zeros((N, H), jnp.float32)
    zeros_w3 = jnp.zeros((N, W3), jnp.float32)
    cn = jnp.asarray(current_node, jnp.int32).reshape(1)

    parts1 = _seg1()(x, src2d, dst2d, ew16, zeros_f)
    h1 = _layer1(parts1, x, W_rel_in, W_root_in, b_rel_in)
    parts2 = _seg2()(h1, src2d, dst2d, ew16, zeros_h)
    tab3 = _layer2(parts2, h1, W_rel_h, W_root_h, b_rel_h, cn)
    parts3 = _seg3()(tab3, src2d, dst2d, ew16, zeros_w3)
    pt, vt = _final(parts3, tab3, W_rel_p, W_root_p, b_rel_p,
                    W_rel_v, W_root_v, b_rel_v)
    return pt.reshape(-1), vt.reshape(-1)


# speedup vs baseline: 3.6541x; 1.0059x over previous
import functools

import jax
import jax.numpy as jnp
from jax import lax
from jax.experimental import pallas as pl
from jax.experimental.pallas import tpu as pltpu
from jax.experimental.pallas import tpu_sc as plsc

N = 10000
E = 320000
F_IN = 128
H = 32
W3 = 40
NC = 2
NS = 16
NW = NC * NS
IW = 125
ROWS = E // IW
RPW = ROWS // NW
EPW = RPW * IW
NPS = N // NS


def _seg_kernel(w_table, nscale, k_chunk, upfront):
    mesh = plsc.VectorSubcoreMesh(core_axis_name="c", subcore_axis_name="s",
                                  num_cores=NC, num_subcores=NS)
    nch = RPW // k_chunk
    ce = k_chunk * IW
    if upfront:
        idx_scr = [
            pltpu.VMEM((RPW, IW), jnp.int32),
            pltpu.VMEM((RPW, IW), jnp.int32),
        ]
    else:
        idx_scr = [
            pltpu.VMEM((k_chunk, IW), jnp.int32),
            pltpu.VMEM((k_chunk, IW), jnp.int32),
        ]
    idx_scr.append(pltpu.VMEM((ce * 16,), jnp.float32))

    @functools.partial(
        pl.kernel,
        out_type=jax.ShapeDtypeStruct((NC, N, w_table), jnp.float32),
        mesh=mesh,
        compiler_params=pltpu.CompilerParams(use_tc_tiling_on_sc=False,
                                             needs_layout_passes=False),
        scratch_types=idx_scr + [
            pltpu.VMEM((ce, w_table), jnp.float32),
            pltpu.VMEM_SHARED((N, w_table), jnp.float32),
            pltpu.SemaphoreType.DMA,
        ],
    )
    def seg(y_hbm, src_hbm, dst_hbm, ew16_hbm, zero_hbm, out_hbm,
            src_v, dst_v, ew16_v, rows_v, acc, sem):
        c = lax.axis_index("c")
        s = lax.axis_index("s")
        w = c * NS + s

        pltpu.sync_copy(zero_hbm.at[pl.ds(s * NPS, NPS)],
                        acc.at[pl.ds(s * NPS, NPS)])

        if upfront:
            pltpu.sync_copy(src_hbm.at[pl.ds(w * RPW, RPW)], src_v)
            pltpu.sync_copy(dst_hbm.at[pl.ds(w * RPW, RPW)], dst_v)

        plsc.subcore_barrier()

        def scale_row(row_base):
            def edge_body(e, _):
                loc = row_base + e
                e16 = ew16_v[pl.ds(loc * 16, 16)]
                ridx = loc + jnp.zeros((16,), jnp.int32)
                for hw in range(nscale // 16):
                    cidx = hw * 16 + lax.iota(jnp.int32, 16)
                    vals = plsc.load_gather(rows_v, [ridx, cidx])
                    plsc.store_scatter(rows_v, [ridx, cidx], vals * e16)
                return 0
            lax.fori_loop(0, IW, edge_body, 0)

        def run_rows(src_ref, dst_ref, row0, n_static):
            cps = [
                pltpu.async_copy(
                    y_hbm.at[src_ref.at[row0 + kk]],
                    rows_v.at[pl.ds(kk * IW, IW)], sem)
                for kk in range(n_static)
            ]
            for cp in cps:
                cp.wait()
            for kk in range(n_static):
                scale_row(kk * IW)
            for kk in range(n_static):
                pltpu.sync_copy(rows_v.at[pl.ds(kk * IW, IW)],
                                acc.at[dst_ref.at[row0 + kk]], add=True)

        def chunk_body(ci, carry):
            e0 = w * EPW + ci * ce
            pltpu.sync_copy(ew16_hbm.at[pl.ds(e0 * 16, ce * 16)], ew16_v)
            if upfront:
                run_rows(src_v, dst_v, ci * k_chunk, k_chunk)
            else:
                r0 = w * RPW + ci * k_chunk
                pltpu.sync_copy(src_hbm.at[pl.ds(r0, k_chunk)], src_v)
                pltpu.sync_copy(dst_hbm.at[pl.ds(r0, k_chunk)], dst_v)
                run_rows(src_v, dst_v, 0, k_chunk)
            return carry
        lax.fori_loop(0, nch, chunk_body, 0)

        plsc.subcore_barrier()
        pltpu.sync_copy(acc.at[pl.ds(s * NPS, NPS)],
                        out_hbm.at[c, pl.ds(s * NPS, NPS)])

    return seg


@functools.lru_cache(maxsize=None)
def _seg1():
    return _seg_kernel(F_IN, F_IN, 2, False)


@functools.lru_cache(maxsize=None)
def _seg2():
    return _seg_kernel(H, H, 8, True)


@functools.lru_cache(maxsize=None)
def _seg3():
    return _seg_kernel(W3, H, 8, True)


_BLK = 2000


def _layer1(parts, x, w_rel, w_root, b_rel):
    def body(p_ref, x_ref, wr_ref, wo_ref, b_ref, h_ref):
        agg = p_ref[0] + p_ref[1]
        h_ref[...] = jnp.maximum(
            jnp.dot(agg, wr_ref[...], preferred_element_type=jnp.float32)
            + b_ref[...]
            + jnp.dot(x_ref[...], wo_ref[...],
                      preferred_element_type=jnp.float32), 0.0)
    return pl.pallas_call(
        body,
        grid=(N // _BLK,),
        in_specs=[
            pl.BlockSpec((NC, _BLK, F_IN), lambda i: (0, i, 0)),
            pl.BlockSpec((_BLK, F_IN), lambda i: (i, 0)),
            pl.BlockSpec((F_IN, H), lambda i: (0, 0)),
            pl.BlockSpec((F_IN, H), lambda i: (0, 0)),
            pl.BlockSpec((1, H), lambda i: (0, 0)),
        ],
        out_specs=pl.BlockSpec((_BLK, H), lambda i: (i, 0)),
        out_shape=jax.ShapeDtypeStruct((N, H), jnp.float32),
    )(parts, x, w_rel, w_root, b_rel.reshape(1, H))


def _layer2(parts, h1, w_rel, w_root, b_rel, cn):
    def body(cn_ref, p_ref, h_ref, wr_ref, wo_ref, b_ref, t_ref):
        i = pl.program_id(0)
        agg = p_ref[0] + p_ref[1]
        h2 = jnp.maximum(
            jnp.dot(agg, wr_ref[...], preferred_element_type=jnp.float32)
            + b_ref[...]
            + jnp.dot(h_ref[...], wo_ref[...],
                      preferred_element_type=jnp.float32), 0.0)
        row = lax.broadcasted_iota(jnp.int32, (_BLK, W3 - H), 0) + i * _BLK
        col = lax.broadcasted_iota(jnp.int32, (_BLK, W3 - H), 1)
        ind = jnp.where((row == cn_ref[0]) & (col == 0), 1.0, 0.0)
        t_ref[...] = jnp.concatenate([h2, ind], axis=1)
    return pl.pallas_call(
        body,
        grid=(N // _BLK,),
        in_specs=[
            pl.BlockSpec(memory_space=pltpu.SMEM),
            pl.BlockSpec((NC, _BLK, H), lambda i: (0, i, 0)),
            pl.BlockSpec((_BLK, H), lambda i: (i, 0)),
            pl.BlockSpec((H, H), lambda i: (0, 0)),
            pl.BlockSpec((H, H), lambda i: (0, 0)),
            pl.BlockSpec((1, H), lambda i: (0, 0)),
        ],
        out_specs=pl.BlockSpec((_BLK, W3), lambda i: (i, 0)),
        out_shape=jax.ShapeDtypeStruct((N, W3), jnp.float32),
    )(cn, parts, h1, w_rel, w_root, b_rel.reshape(1, H))


def _final(parts, tab, w_rel_p, w_root_p, b_p, w_rel_v, w_root_v, b_v):
    def body(p_ref, t_ref, wrp_ref, wop_ref, bp_ref, wrv_ref, wov_ref,
             bv_ref, pt_ref, vt_ref):
        ssum = p_ref[0] + p_ref[1]
        agg = ssum[:, :H]
        cnt = ssum[:, H:H + 1]
        h2 = t_ref[:, :H]
        p = (jnp.dot(agg, wrp_ref[...], preferred_element_type=jnp.float32)
             + bp_ref[0]
             + jnp.dot(h2, wop_ref[...], preferred_element_type=jnp.float32))
        v = (jnp.dot(agg, wrv_ref[...], preferred_element_type=jnp.float32)
             + bv_ref[0]
             + jnp.dot(h2, wov_ref[...], preferred_element_type=jnp.float32))
        mask = jnp.where(cnt > 0.0, 1.0, 0.0)
        pt = mask * p
        pt = jnp.where(pt == 0.0, -jnp.inf, pt)
        m = jnp.max(pt)
        ex = jnp.exp(pt - m)
        pt_ref[...] = ex / jnp.sum(ex)
        vt_ref[...] = mask * v
    return pl.pallas_call(
        body,
        in_specs=[
            pl.BlockSpec((NC, N, W3), lambda: (0, 0, 0)),
            pl.BlockSpec((N, W3), lambda: (0, 0)),
            pl.BlockSpec((H, 1), lambda: (0, 0)),
            pl.BlockSpec((H, 1), lambda: (0, 0)),
            pl.BlockSpec(memory_space=pltpu.SMEM),
            pl.BlockSpec((H, 1), lambda: (0, 0)),
            pl.BlockSpec((H, 1), lambda: (0, 0)),
            pl.BlockSpec(memory_space=pltpu.SMEM),
        ],
        out_specs=[
            pl.BlockSpec((N, 1), lambda: (0, 0)),
            pl.BlockSpec((N, 1), lambda: (0, 0)),
        ],
        out_shape=[jax.ShapeDtypeStruct((N, 1), jnp.float32)] * 2,
    )(parts, tab, w_rel_p, w_root_p, b_p, w_rel_v, w_root_v, b_v)


def kernel(x, edge_index, edge_attr, current_node,
           W_rel_in, b_rel_in, W_root_in,
           W_rel_h, b_rel_h, W_root_h,
           W_rel_p, b_rel_p, W_root_p,
           W_rel_v, b_rel_v, W_root_v):
    src2d = edge_index[0].reshape(ROWS, IW)
    dst2d = edge_index[1].reshape(ROWS, IW)
    ew16 = jnp.broadcast_to(edge_attr.reshape(E, 1), (E, 16)).reshape(E * 16)
    zeros_f = jnp.zeros((N, F_IN), jnp.float32)
    zeros_h = jnp.zeros((N, H), jnp.float32)
    zeros_w3 = jnp.zeros((N, W3), jnp.float32)
    cn = jnp.asarray(current_node, jnp.int32).reshape(1)

    parts1 = _seg1()(x, src2d, dst2d, ew16, zeros_f)
    h1 = _layer1(parts1, x, W_rel_in, W_root_in, b_rel_in)
    parts2 = _seg2()(h1, src2d, dst2d, ew16, zeros_h)
    tab3 = _layer2(parts2, h1, W_rel_h, W_root_h, b_rel_h, cn)
    parts3 = _seg3()(tab3, src2d, dst2d, ew16, zeros_w3)
    pt, vt = _final(parts3, tab3, W_rel_p, W_root_p, b_rel_p,
                    W_rel_v, W_root_v, b_rel_v)
    return pt.reshape(-1), vt.reshape(-1)
